# baseline (speedup 1.0000x reference)
import functools

import jax
import jax.numpy as jnp
from jax import lax
from jax.experimental import pallas as pl
from jax.experimental.pallas import tpu as pltpu
from jax.experimental.pallas import tpu_sc as plsc

K_NN = 10
GS = 128
QT = 256
KT = 2048
GT = KT // GS
NEG = float("-inf")
BIG = 2**30


def _p1_body(n_keys, ki_grid, q_ref, k_ref, s_ref, m_ref):
    ki = pl.program_id(0)
    scores = lax.dot_general(
        q_ref[...], k_ref[...], (((1,), (1,)), ((), ())),
        preferred_element_type=jnp.float32)
    for j in range(GT):
        s_ref[j] = scores[:, j * GS:(j + 1) * GS]

    def store_gmax(x):
        gmax = jnp.concatenate(
            [jnp.max(x[:, j * GS:(j + 1) * GS], axis=1, keepdims=True)
             for j in range(GT)], axis=1)
        m_ref[...] = jnp.transpose(gmax)

    @pl.when(ki < ki_grid - 1)
    def _():
        store_gmax(scores)

    @pl.when(ki == ki_grid - 1)
    def _():
        col_iota = lax.broadcasted_iota(jnp.int32, (QT, KT), 1)
        store_gmax(jnp.where(col_iota + ki * KT < n_keys, scores, NEG))


def _p1b_body(nq, m_ref, r_ref, g_ref):
    qi = pl.program_id(0)
    m = m_ref[...]
    mh = m.shape[0]
    gids = lax.broadcasted_iota(jnp.int32, (mh, QT), 0)
    picks = []
    for _ in range(K_NN):
        mx = jnp.max(m, axis=0, keepdims=True)
        g = jnp.min(jnp.where(m == mx, gids, BIG), axis=0, keepdims=True)
        m = jnp.where(gids == g, NEG, m)
        picks.append(g)
    picks.extend([picks[-1]] * (16 - K_NN))
    gmat = jnp.concatenate(picks, axis=0)
    qlane = qi * QT + lax.broadcasted_iota(jnp.int32, (1, QT), 1)
    r_ref[...] = gmat * nq + qlane
    g_ref[...] = jnp.transpose(gmat)


def _sc_gather_body(chunks, ccpr, table_hbm, idx_hbm, out_hbm,
                    idx_v, rows_v, sem):
    wid = lax.axis_index("s") * 2 + lax.axis_index("c")
    j0 = (wid * chunks) // ccpr
    cc0 = (wid * chunks) % ccpr
    pltpu.sync_copy(idx_hbm.at[j0, pl.ds(cc0, chunks)], idx_v)
    for c in range(chunks):
        pltpu.async_copy(table_hbm.at[idx_v.at[c]], rows_v, sem).wait()
        pltpu.sync_copy(rows_v,
                        out_hbm.at[pl.ds((wid * chunks + c) * 128, 128)])


def _p3_body(n_keys, c_ref, g_ref, s_out, i_out):
    cand = jnp.concatenate([c_ref[j] for j in range(K_NN)], axis=1)
    g = g_ref[...]
    lane = lax.broadcasted_iota(jnp.int32, (QT, GS), 1)
    idx = jnp.concatenate(
        [g[:, j:j + 1] * GS + lane for j in range(K_NN)], axis=1)
    cand = jnp.where(idx < n_keys, cand, NEG)
    svals, ivals = [], []
    for _ in range(K_NN):
        mx = jnp.max(cand, axis=1, keepdims=True)
        best = jnp.min(jnp.where(cand == mx, idx, BIG), axis=1,
                       keepdims=True)
        cand = jnp.where(idx == best, NEG, cand)
        svals.append(mx)
        ivals.append(best)
    s_out[...] = jnp.concatenate(svals, axis=1)
    i_out[...] = jnp.concatenate(ivals, axis=1).astype(jnp.int32)


def kernel(queries, keys):
    nq, d = queries.shape
    n_keys = keys.shape[0]
    ki_grid = -(-n_keys // KT)
    kp = ki_grid * KT
    n_groups = kp // GS
    qi_grid = nq // QT

    s_full, m = pl.pallas_call(
        functools.partial(_p1_body, n_keys, ki_grid),
        grid=(ki_grid, qi_grid),
        in_specs=[
            pl.BlockSpec((QT, d), lambda ki, qi: (qi, 0)),
            pl.BlockSpec((KT, d), lambda ki, qi: (ki, 0)),
        ],
        out_specs=[
            pl.BlockSpec((GT, QT, GS), lambda ki, qi: (ki, qi, 0)),
            pl.BlockSpec((GT, QT), lambda ki, qi: (ki, qi)),
        ],
        out_shape=[
            jax.ShapeDtypeStruct((n_groups, nq, GS), jnp.float32),
            jax.ShapeDtypeStruct((n_groups, nq), jnp.float32),
        ],
    )(queries, keys)

    rowids_t, gq = pl.pallas_call(
        functools.partial(_p1b_body, nq),
        grid=(qi_grid,),
        in_specs=[pl.BlockSpec((n_groups, QT), lambda qi: (0, qi))],
        out_specs=[
            pl.BlockSpec((16, QT), lambda qi: (0, qi)),
            pl.BlockSpec((QT, 16), lambda qi: (qi, 0)),
        ],
        out_shape=[
            jax.ShapeDtypeStruct((16, nq), jnp.int32),
            jax.ShapeDtypeStruct((nq, 16), jnp.int32),
        ],
    )(m)

    n_rows = nq * 16
    chunks = n_rows // (32 * 128)
    ccpr = nq // 128
    mesh = plsc.VectorSubcoreMesh(core_axis_name="c", subcore_axis_name="s")
    gathered = pl.kernel(
        functools.partial(_sc_gather_body, chunks, ccpr),
        mesh=mesh,
        out_type=jax.ShapeDtypeStruct((n_rows, GS), jnp.float32),
        scratch_types=[
            pltpu.VMEM((chunks, 128), jnp.int32),
            pltpu.VMEM((128, GS), jnp.float32),
            pltpu.SemaphoreType.DMA,
        ],
    )(s_full.reshape(n_groups * nq, GS),
      rowids_t.reshape(16, ccpr, 128))

    return pl.pallas_call(
        functools.partial(_p3_body, n_keys),
        grid=(qi_grid,),
        in_specs=[
            pl.BlockSpec((K_NN, QT, GS), lambda qi: (0, qi, 0)),
            pl.BlockSpec((QT, 16), lambda qi: (qi, 0)),
        ],
        out_specs=[
            pl.BlockSpec((QT, K_NN), lambda qi: (qi, 0)),
            pl.BlockSpec((QT, K_NN), lambda qi: (qi, 0)),
        ],
        out_shape=[
            jax.ShapeDtypeStruct((nq, K_NN), jnp.float32),
            jax.ShapeDtypeStruct((nq, K_NN), jnp.int32),
        ],
    )(gathered.reshape(16, nq, GS), gq)


# speedup vs baseline: 1.0348x; 1.0348x over previous
import functools

import jax
import jax.numpy as jnp
from jax import lax
from jax.experimental import pallas as pl
from jax.experimental.pallas import tpu as pltpu
from jax.experimental.pallas import tpu_sc as plsc

K_NN = 10
GS = 128
QT = 256
KT = 2048
GT = KT // GS
NEG = float("-inf")
BIG = 2**30


def _p1_body(n_keys, ki_grid, q_ref, k_ref, s_ref, m_ref):
    ki = pl.program_id(0)
    scores = lax.dot_general(
        q_ref[...], k_ref[...], (((1,), (1,)), ((), ())),
        preferred_element_type=jnp.float32)
    for j in range(GT):
        s_ref[j] = scores[:, j * GS:(j + 1) * GS]

    col_iota = lax.broadcasted_iota(jnp.int32, (QT, KT), 1)
    masked = jnp.where(col_iota + ki * KT < n_keys, scores, NEG)
    gmax = jnp.concatenate(
        [jnp.max(masked[:, j * GS:(j + 1) * GS], axis=1, keepdims=True)
         for j in range(GT)], axis=1)
    m_ref[...] = jnp.transpose(gmax)
    del ki_grid


def _p1b_body(nq, m_ref, r_ref, g_ref):
    qi = pl.program_id(0)
    m = m_ref[...]
    mh = m.shape[0]
    gids = lax.broadcasted_iota(jnp.int32, (mh, QT), 0)
    picks = []
    for _ in range(K_NN):
        mx = jnp.max(m, axis=0, keepdims=True)
        g = jnp.min(jnp.where(m == mx, gids, BIG), axis=0, keepdims=True)
        m = jnp.where(gids == g, NEG, m)
        picks.append(g)
    picks.extend([picks[-1]] * (16 - K_NN))
    gmat = jnp.concatenate(picks, axis=0)
    qlane = qi * QT + lax.broadcasted_iota(jnp.int32, (1, QT), 1)
    r_ref[...] = gmat * nq + qlane
    g_ref[...] = jnp.transpose(gmat)


def _sc_gather_body(chunks, ccpr, table_hbm, idx_hbm, out_hbm,
                    idx_v, rows_v, sem):
    wid = lax.axis_index("s") * 2 + lax.axis_index("c")
    j0 = (wid * chunks) // ccpr
    cc0 = (wid * chunks) % ccpr
    pltpu.sync_copy(idx_hbm.at[j0, pl.ds(cc0, chunks)], idx_v)
    for c in range(chunks):
        pltpu.async_copy(table_hbm.at[idx_v.at[c]], rows_v, sem).wait()
        pltpu.sync_copy(rows_v,
                        out_hbm.at[pl.ds((wid * chunks + c) * 128, 128)])


def _p3_body(n_keys, c_ref, g_ref, s_out, i_out):
    cand = jnp.concatenate([c_ref[j] for j in range(K_NN)], axis=1)
    g = g_ref[...]
    lane = lax.broadcasted_iota(jnp.int32, (QT, GS), 1)
    idx = jnp.concatenate(
        [g[:, j:j + 1] * GS + lane for j in range(K_NN)], axis=1)
    cand = jnp.where(idx < n_keys, cand, NEG)
    svals, ivals = [], []
    for _ in range(K_NN):
        mx = jnp.max(cand, axis=1, keepdims=True)
        best = jnp.min(jnp.where(cand == mx, idx, BIG), axis=1,
                       keepdims=True)
        cand = jnp.where(idx == best, NEG, cand)
        svals.append(mx)
        ivals.append(best)
    s_out[...] = jnp.concatenate(svals, axis=1)
    i_out[...] = jnp.concatenate(ivals, axis=1).astype(jnp.int32)


def kernel(queries, keys):
    nq, d = queries.shape
    n_keys = keys.shape[0]
    ki_grid = -(-n_keys // KT)
    kp = ki_grid * KT
    n_groups = kp // GS
    qi_grid = nq // QT

    s_full, m = pl.pallas_call(
        functools.partial(_p1_body, n_keys, ki_grid),
        grid=(ki_grid, qi_grid),
        in_specs=[
            pl.BlockSpec((QT, d), lambda ki, qi: (qi, 0)),
            pl.BlockSpec((KT, d), lambda ki, qi: (ki, 0)),
        ],
        out_specs=[
            pl.BlockSpec((GT, QT, GS), lambda ki, qi: (ki, qi, 0)),
            pl.BlockSpec((GT, QT), lambda ki, qi: (ki, qi)),
        ],
        out_shape=[
            jax.ShapeDtypeStruct((n_groups, nq, GS), jnp.float32),
            jax.ShapeDtypeStruct((n_groups, nq), jnp.float32),
        ],
    )(queries, keys)

    rowids_t, gq = pl.pallas_call(
        functools.partial(_p1b_body, nq),
        grid=(qi_grid,),
        in_specs=[pl.BlockSpec((n_groups, QT), lambda qi: (0, qi))],
        out_specs=[
            pl.BlockSpec((16, QT), lambda qi: (0, qi)),
            pl.BlockSpec((QT, 16), lambda qi: (qi, 0)),
        ],
        out_shape=[
            jax.ShapeDtypeStruct((16, nq), jnp.int32),
            jax.ShapeDtypeStruct((nq, 16), jnp.int32),
        ],
    )(m)

    n_rows = nq * 16
    chunks = n_rows // (32 * 128)
    ccpr = nq // 128
    mesh = plsc.VectorSubcoreMesh(core_axis_name="c", subcore_axis_name="s")
    gathered = pl.kernel(
        functools.partial(_sc_gather_body, chunks, ccpr),
        mesh=mesh,
        out_type=jax.ShapeDtypeStruct((n_rows, GS), jnp.float32),
        scratch_types=[
            pltpu.VMEM((chunks, 128), jnp.int32),
            pltpu.VMEM((128, GS), jnp.float32),
            pltpu.SemaphoreType.DMA,
        ],
    )(s_full.reshape(n_groups * nq, GS),
      rowids_t.reshape(16, ccpr, 128))

    return pl.pallas_call(
        functools.partial(_p3_body, n_keys),
        grid=(qi_grid,),
        in_specs=[
            pl.BlockSpec((K_NN, QT, GS), lambda qi: (0, qi, 0)),
            pl.BlockSpec((QT, 16), lambda qi: (qi, 0)),
        ],
        out_specs=[
            pl.BlockSpec((QT, K_NN), lambda qi: (qi, 0)),
            pl.BlockSpec((QT, K_NN), lambda qi: (qi, 0)),
        ],
        out_shape=[
            jax.ShapeDtypeStruct((nq, K_NN), jnp.float32),
            jax.ShapeDtypeStruct((nq, K_NN), jnp.int32),
        ],
    )(gathered.reshape(16, nq, GS), gq)


# speedup vs baseline: 1.0434x; 1.0083x over previous
import functools

import jax
import jax.numpy as jnp
from jax import lax
from jax.experimental import pallas as pl
from jax.experimental.pallas import tpu as pltpu
from jax.experimental.pallas import tpu_sc as plsc

K_NN = 10
GS = 128
QT = 256
KT = 2048
GT = KT // GS
NEG = float("-inf")
BIG = 2**30


def _p1_body(n_keys, ki_grid, q_ref, k_ref, s_ref, m_ref):
    ki = pl.program_id(0)
    scores = lax.dot_general(
        q_ref[...], k_ref[...], (((1,), (1,)), ((), ())),
        preferred_element_type=jnp.float32)
    for j in range(GT):
        s_ref[j] = scores[:, j * GS:(j + 1) * GS]

    col_iota = lax.broadcasted_iota(jnp.int32, (QT, KT), 1)
    masked = jnp.where(col_iota + ki * KT < n_keys, scores, NEG)
    gmax = jnp.concatenate(
        [jnp.max(masked[:, j * GS:(j + 1) * GS], axis=1, keepdims=True)
         for j in range(GT)], axis=1)
    m_ref[...] = jnp.transpose(gmax)
    del ki_grid


def _p1b_body(nq, m_ref, r_ref, g_ref):
    qi = pl.program_id(0)
    m = m_ref[...]
    mh = m.shape[0]
    gids = lax.broadcasted_iota(jnp.int32, (mh, QT), 0)
    picks = []
    for _ in range(K_NN):
        mx = jnp.max(m, axis=0, keepdims=True)
        g = jnp.min(jnp.where(m == mx, gids, BIG), axis=0, keepdims=True)
        m = jnp.where(gids == g, NEG, m)
        picks.append(g)
    picks.extend([picks[-1]] * (16 - K_NN))
    gmat = jnp.concatenate(picks, axis=0)
    qlane = qi * QT + lax.broadcasted_iota(jnp.int32, (1, QT), 1)
    r_ref[...] = gmat * nq + qlane
    g_ref[...] = jnp.transpose(gmat)


def _sc_gather_body(chunks, ccpr, table_hbm, idx_hbm, out_hbm,
                    idx_v, rows_v, sem0, sem1):
    wid = lax.axis_index("s") * 2 + lax.axis_index("c")
    j0 = (wid * chunks) // ccpr
    cc0 = (wid * chunks) % ccpr
    pltpu.sync_copy(idx_hbm.at[j0, pl.ds(cc0, chunks)], idx_v)
    sems = [sem0, sem1]
    copies = [None, None]
    copies[0] = pltpu.async_copy(
        table_hbm.at[idx_v.at[0]], rows_v.at[0], sems[0])
    for c in range(chunks):
        b = c % 2
        if c + 1 < chunks:
            copies[(c + 1) % 2] = pltpu.async_copy(
                table_hbm.at[idx_v.at[c + 1]], rows_v.at[(c + 1) % 2],
                sems[(c + 1) % 2])
        copies[b].wait()
        pltpu.sync_copy(rows_v.at[b],
                        out_hbm.at[pl.ds((wid * chunks + c) * 128, 128)])


def _p3_body(n_keys, c_ref, g_ref, s_out, i_out):
    cand = jnp.concatenate([c_ref[j] for j in range(K_NN)], axis=1)
    g = g_ref[...]
    lane = lax.broadcasted_iota(jnp.int32, (QT, GS), 1)
    idx = jnp.concatenate(
        [g[:, j:j + 1] * GS + lane for j in range(K_NN)], axis=1)
    cand = jnp.where(idx < n_keys, cand, NEG)
    svals, ivals = [], []
    for _ in range(K_NN):
        mx = jnp.max(cand, axis=1, keepdims=True)
        best = jnp.min(jnp.where(cand == mx, idx, BIG), axis=1,
                       keepdims=True)
        cand = jnp.where(idx == best, NEG, cand)
        svals.append(mx)
        ivals.append(best)
    s_out[...] = jnp.concatenate(svals, axis=1)
    i_out[...] = jnp.concatenate(ivals, axis=1).astype(jnp.int32)


def kernel(queries, keys):
    nq, d = queries.shape
    n_keys = keys.shape[0]
    ki_grid = -(-n_keys // KT)
    kp = ki_grid * KT
    n_groups = kp // GS
    qi_grid = nq // QT

    s_full, m = pl.pallas_call(
        functools.partial(_p1_body, n_keys, ki_grid),
        grid=(ki_grid, qi_grid),
        in_specs=[
            pl.BlockSpec((QT, d), lambda ki, qi: (qi, 0)),
            pl.BlockSpec((KT, d), lambda ki, qi: (ki, 0)),
        ],
        out_specs=[
            pl.BlockSpec((GT, QT, GS), lambda ki, qi: (ki, qi, 0)),
            pl.BlockSpec((GT, QT), lambda ki, qi: (ki, qi)),
        ],
        out_shape=[
            jax.ShapeDtypeStruct((n_groups, nq, GS), jnp.float32),
            jax.ShapeDtypeStruct((n_groups, nq), jnp.float32),
        ],
    )(queries, keys)

    rowids_t, gq = pl.pallas_call(
        functools.partial(_p1b_body, nq),
        grid=(qi_grid,),
        in_specs=[pl.BlockSpec((n_groups, QT), lambda qi: (0, qi))],
        out_specs=[
            pl.BlockSpec((16, QT), lambda qi: (0, qi)),
            pl.BlockSpec((QT, 16), lambda qi: (qi, 0)),
        ],
        out_shape=[
            jax.ShapeDtypeStruct((16, nq), jnp.int32),
            jax.ShapeDtypeStruct((nq, 16), jnp.int32),
        ],
    )(m)

    jrows = 16
    n_rows = nq * jrows
    chunks = n_rows // (32 * 128)
    ccpr = nq // 128
    mesh = plsc.VectorSubcoreMesh(core_axis_name="c", subcore_axis_name="s")
    gathered = pl.kernel(
        functools.partial(_sc_gather_body, chunks, ccpr),
        mesh=mesh,
        out_type=jax.ShapeDtypeStruct((n_rows, GS), jnp.float32),
        scratch_types=[
            pltpu.VMEM((chunks, 128), jnp.int32),
            pltpu.VMEM((2, 128, GS), jnp.float32),
            pltpu.SemaphoreType.DMA,
            pltpu.SemaphoreType.DMA,
        ],
    )(s_full.reshape(n_groups * nq, GS),
      rowids_t.reshape(16, ccpr, 128)[:jrows])

    return pl.pallas_call(
        functools.partial(_p3_body, n_keys),
        grid=(qi_grid,),
        in_specs=[
            pl.BlockSpec((K_NN, QT, GS), lambda qi: (0, qi, 0)),
            pl.BlockSpec((QT, 16), lambda qi: (qi, 0)),
        ],
        out_specs=[
            pl.BlockSpec((QT, K_NN), lambda qi: (qi, 0)),
            pl.BlockSpec((QT, K_NN), lambda qi: (qi, 0)),
        ],
        out_shape=[
            jax.ShapeDtypeStruct((nq, K_NN), jnp.float32),
            jax.ShapeDtypeStruct((nq, K_NN), jnp.int32),
        ],
    )(gathered.reshape(jrows, nq, GS), gq)


# speedup vs baseline: 1.3254x; 1.2704x over previous
import functools

import jax
import jax.numpy as jnp
from jax import lax
from jax.experimental import pallas as pl
from jax.experimental.pallas import tpu as pltpu
from jax.experimental.pallas import tpu_sc as plsc

K_NN = 10
GS = 128
QT = 512
KT = 2048
GT = KT // GS
NEG = float("-inf")
BIG = 2**30


def _p1_body(n_keys, ki_grid, q_ref, k_ref, s_ref, m_ref):
    ki = pl.program_id(0)
    scores = lax.dot_general(
        q_ref[...], k_ref[...], (((1,), (1,)), ((), ())),
        preferred_element_type=jnp.float32)
    for j in range(GT):
        s_ref[j] = scores[:, j * GS:(j + 1) * GS]

    qt = q_ref.shape[0]
    col_iota = lax.broadcasted_iota(jnp.int32, (qt, KT), 1)
    masked = jnp.where(col_iota + ki * KT < n_keys, scores, NEG)
    gmax = jnp.concatenate(
        [jnp.max(masked[:, j * GS:(j + 1) * GS], axis=1, keepdims=True)
         for j in range(GT)], axis=1)
    m_ref[...] = jnp.transpose(gmax)
    del ki_grid


def _p1b_body(nq, m_ref, r_ref, g_ref):
    qi = pl.program_id(0)
    m = m_ref[...]
    mh = m.shape[0]
    qt = m.shape[1]
    gids = lax.broadcasted_iota(jnp.int32, (mh, qt), 0)
    picks = []
    for _ in range(K_NN):
        mx = jnp.max(m, axis=0, keepdims=True)
        g = jnp.min(jnp.where(m == mx, gids, BIG), axis=0, keepdims=True)
        m = jnp.where(gids == g, NEG, m)
        picks.append(g)
    picks.extend([picks[-1]] * (16 - K_NN))
    gmat = jnp.concatenate(picks, axis=0)
    qlane = qi * qt + lax.broadcasted_iota(jnp.int32, (1, qt), 1)
    r_ref[...] = gmat * nq + qlane
    g_ref[...] = jnp.transpose(gmat)


def _sc_gather_body(chunks, ccpr, table_hbm, idx_hbm, out_hbm,
                    idx_v, rows_v, sem0, sem1):
    wid = lax.axis_index("s") * 2 + lax.axis_index("c")
    j0 = (wid * chunks) // ccpr
    cc0 = (wid * chunks) % ccpr
    pltpu.sync_copy(idx_hbm.at[j0, pl.ds(cc0, chunks)], idx_v)
    sems = [sem0, sem1]
    copies = [None, None]
    copies[0] = pltpu.async_copy(
        table_hbm.at[idx_v.at[0]], rows_v.at[0], sems[0])
    for c in range(chunks):
        b = c % 2
        if c + 1 < chunks:
            copies[(c + 1) % 2] = pltpu.async_copy(
                table_hbm.at[idx_v.at[c + 1]], rows_v.at[(c + 1) % 2],
                sems[(c + 1) % 2])
        copies[b].wait()
        pltpu.sync_copy(rows_v.at[b],
                        out_hbm.at[pl.ds((wid * chunks + c) * 128, 128)])


def _p3_body(n_keys, c_ref, g_ref, s_out, i_out):
    cand = jnp.concatenate([c_ref[j] for j in range(K_NN)], axis=1)
    g = g_ref[...]
    qt = g.shape[0]
    lane = lax.broadcasted_iota(jnp.int32, (qt, GS), 1)
    idx = jnp.concatenate(
        [g[:, j:j + 1] * GS + lane for j in range(K_NN)], axis=1)
    cand = jnp.where(idx < n_keys, cand, NEG)
    svals, ivals = [], []
    for _ in range(K_NN):
        mx = jnp.max(cand, axis=1, keepdims=True)
        best = jnp.min(jnp.where(cand == mx, idx, BIG), axis=1,
                       keepdims=True)
        cand = jnp.where(idx == best, NEG, cand)
        svals.append(mx)
        ivals.append(best)
    s_out[...] = jnp.concatenate(svals, axis=1)
    i_out[...] = jnp.concatenate(ivals, axis=1).astype(jnp.int32)


def kernel(queries, keys):
    nq, d = queries.shape
    n_keys = keys.shape[0]
    ki_grid = -(-n_keys // KT)
    kp = ki_grid * KT
    n_groups = kp // GS
    qi_grid = nq // QT

    s_full, m = pl.pallas_call(
        functools.partial(_p1_body, n_keys, ki_grid),
        grid=(ki_grid, qi_grid),
        in_specs=[
            pl.BlockSpec((QT, d), lambda ki, qi: (qi, 0)),
            pl.BlockSpec((KT, d), lambda ki, qi: (ki, 0)),
        ],
        out_specs=[
            pl.BlockSpec((GT, QT, GS), lambda ki, qi: (ki, qi, 0)),
            pl.BlockSpec((GT, QT), lambda ki, qi: (ki, qi)),
        ],
        out_shape=[
            jax.ShapeDtypeStruct((n_groups, nq, GS), jnp.float32),
            jax.ShapeDtypeStruct((n_groups, nq), jnp.float32),
        ],
    )(queries, keys)

    rowids_t, gq = pl.pallas_call(
        functools.partial(_p1b_body, nq),
        grid=(qi_grid,),
        in_specs=[pl.BlockSpec((n_groups, QT), lambda qi: (0, qi))],
        out_specs=[
            pl.BlockSpec((16, QT), lambda qi: (0, qi)),
            pl.BlockSpec((QT, 16), lambda qi: (qi, 0)),
        ],
        out_shape=[
            jax.ShapeDtypeStruct((16, nq), jnp.int32),
            jax.ShapeDtypeStruct((nq, 16), jnp.int32),
        ],
    )(m)

    jrows = 16
    n_rows = nq * jrows
    chunks = n_rows // (32 * 128)
    ccpr = nq // 128
    mesh = plsc.VectorSubcoreMesh(core_axis_name="c", subcore_axis_name="s")
    gathered = pl.kernel(
        functools.partial(_sc_gather_body, chunks, ccpr),
        mesh=mesh,
        out_type=jax.ShapeDtypeStruct((n_rows, GS), jnp.float32),
        scratch_types=[
            pltpu.VMEM((chunks, 128), jnp.int32),
            pltpu.VMEM((2, 128, GS), jnp.float32),
            pltpu.SemaphoreType.DMA,
            pltpu.SemaphoreType.DMA,
        ],
    )(s_full.reshape(n_groups * nq, GS),
      rowids_t.reshape(16, ccpr, 128)[:jrows])

    return pl.pallas_call(
        functools.partial(_p3_body, n_keys),
        grid=(qi_grid,),
        in_specs=[
            pl.BlockSpec((K_NN, QT, GS), lambda qi: (0, qi, 0)),
            pl.BlockSpec((QT, 16), lambda qi: (qi, 0)),
        ],
        out_specs=[
            pl.BlockSpec((QT, K_NN), lambda qi: (qi, 0)),
            pl.BlockSpec((QT, K_NN), lambda qi: (qi, 0)),
        ],
        out_shape=[
            jax.ShapeDtypeStruct((nq, K_NN), jnp.float32),
            jax.ShapeDtypeStruct((nq, K_NN), jnp.int32),
        ],
    )(gathered.reshape(jrows, nq, GS), gq)


# speedup vs baseline: 1.5443x; 1.1651x over previous
import functools

import jax
import jax.numpy as jnp
from jax import lax
from jax.experimental import pallas as pl
from jax.experimental.pallas import tpu as pltpu
from jax.experimental.pallas import tpu_sc as plsc

K_NN = 10
GS = 128
QT = 1024
KT = 2048
GT = KT // GS
NEG = float("-inf")
BIG = 2**30


def _p1_body(n_keys, ki_grid, q_ref, k_ref, s_ref, m_ref):
    ki = pl.program_id(0)
    scores = lax.dot_general(
        q_ref[...], k_ref[...], (((1,), (1,)), ((), ())),
        preferred_element_type=jnp.float32)
    for j in range(GT):
        s_ref[j] = scores[:, j * GS:(j + 1) * GS]

    qt = q_ref.shape[0]
    col_iota = lax.broadcasted_iota(jnp.int32, (qt, KT), 1)
    masked = jnp.where(col_iota + ki * KT < n_keys, scores, NEG)
    gmax = jnp.concatenate(
        [jnp.max(masked[:, j * GS:(j + 1) * GS], axis=1, keepdims=True)
         for j in range(GT)], axis=1)
    m_ref[...] = jnp.transpose(gmax)
    del ki_grid


def _p1b_body(nq, m_ref, r_ref, g_ref):
    qi = pl.program_id(0)
    m = m_ref[...]
    mh = m.shape[0]
    qt = m.shape[1]
    gids = lax.broadcasted_iota(jnp.int32, (mh, qt), 0)
    picks = []
    for _ in range(K_NN):
        mx = jnp.max(m, axis=0, keepdims=True)
        g = jnp.min(jnp.where(m == mx, gids, BIG), axis=0, keepdims=True)
        m = jnp.where(gids == g, NEG, m)
        picks.append(g)
    picks.extend([picks[-1]] * (16 - K_NN))
    gmat = jnp.concatenate(picks, axis=0)
    qlane = qi * qt + lax.broadcasted_iota(jnp.int32, (1, qt), 1)
    r_ref[...] = gmat * nq + qlane
    g_ref[...] = jnp.transpose(gmat)


def _sc_gather_body(chunks, ccpr, table_hbm, idx_hbm, out_hbm,
                    idx_v, rows_v, sem0, sem1):
    wid = lax.axis_index("s") * 2 + lax.axis_index("c")
    j0 = (wid * chunks) // ccpr
    cc0 = (wid * chunks) % ccpr
    pltpu.sync_copy(idx_hbm.at[j0, pl.ds(cc0, chunks)], idx_v)
    sems = [sem0, sem1]
    copies = [None, None]
    copies[0] = pltpu.async_copy(
        table_hbm.at[idx_v.at[0]], rows_v.at[0], sems[0])
    for c in range(chunks):
        b = c % 2
        if c + 1 < chunks:
            copies[(c + 1) % 2] = pltpu.async_copy(
                table_hbm.at[idx_v.at[c + 1]], rows_v.at[(c + 1) % 2],
                sems[(c + 1) % 2])
        copies[b].wait()
        pltpu.sync_copy(rows_v.at[b],
                        out_hbm.at[pl.ds((wid * chunks + c) * 128, 128)])


def _p3_body(n_keys, c_ref, g_ref, s_out, i_out):
    cand = jnp.concatenate([c_ref[j] for j in range(K_NN)], axis=1)
    g = g_ref[...]
    qt = g.shape[0]
    lane = lax.broadcasted_iota(jnp.int32, (qt, GS), 1)
    idx = jnp.concatenate(
        [g[:, j:j + 1] * GS + lane for j in range(K_NN)], axis=1)
    cand = jnp.where(idx < n_keys, cand, NEG)
    svals, ivals = [], []
    for _ in range(K_NN):
        mx = jnp.max(cand, axis=1, keepdims=True)
        best = jnp.min(jnp.where(cand == mx, idx, BIG), axis=1,
                       keepdims=True)
        cand = jnp.where(idx == best, NEG, cand)
        svals.append(mx)
        ivals.append(best)
    s_out[...] = jnp.concatenate(svals, axis=1)
    i_out[...] = jnp.concatenate(ivals, axis=1).astype(jnp.int32)


def kernel(queries, keys):
    nq, d = queries.shape
    n_keys = keys.shape[0]
    ki_grid = -(-n_keys // KT)
    kp = ki_grid * KT
    n_groups = kp // GS
    qi_grid = nq // QT

    s_full, m = pl.pallas_call(
        functools.partial(_p1_body, n_keys, ki_grid),
        grid=(ki_grid, qi_grid),
        in_specs=[
            pl.BlockSpec((QT, d), lambda ki, qi: (qi, 0)),
            pl.BlockSpec((KT, d), lambda ki, qi: (ki, 0)),
        ],
        out_specs=[
            pl.BlockSpec((GT, QT, GS), lambda ki, qi: (ki, qi, 0)),
            pl.BlockSpec((GT, QT), lambda ki, qi: (ki, qi)),
        ],
        out_shape=[
            jax.ShapeDtypeStruct((n_groups, nq, GS), jnp.float32),
            jax.ShapeDtypeStruct((n_groups, nq), jnp.float32),
        ],
    )(queries, keys)

    rowids_t, gq = pl.pallas_call(
        functools.partial(_p1b_body, nq),
        grid=(qi_grid,),
        in_specs=[pl.BlockSpec((n_groups, QT), lambda qi: (0, qi))],
        out_specs=[
            pl.BlockSpec((16, QT), lambda qi: (0, qi)),
            pl.BlockSpec((QT, 16), lambda qi: (qi, 0)),
        ],
        out_shape=[
            jax.ShapeDtypeStruct((16, nq), jnp.int32),
            jax.ShapeDtypeStruct((nq, 16), jnp.int32),
        ],
    )(m)

    jrows = 16
    n_rows = nq * jrows
    chunks = n_rows // (32 * 128)
    ccpr = nq // 128
    mesh = plsc.VectorSubcoreMesh(core_axis_name="c", subcore_axis_name="s")
    gathered = pl.kernel(
        functools.partial(_sc_gather_body, chunks, ccpr),
        mesh=mesh,
        out_type=jax.ShapeDtypeStruct((n_rows, GS), jnp.float32),
        scratch_types=[
            pltpu.VMEM((chunks, 128), jnp.int32),
            pltpu.VMEM((2, 128, GS), jnp.float32),
            pltpu.SemaphoreType.DMA,
            pltpu.SemaphoreType.DMA,
        ],
    )(s_full.reshape(n_groups * nq, GS),
      rowids_t.reshape(16, ccpr, 128)[:jrows])

    return pl.pallas_call(
        functools.partial(_p3_body, n_keys),
        grid=(qi_grid,),
        in_specs=[
            pl.BlockSpec((K_NN, QT, GS), lambda qi: (0, qi, 0)),
            pl.BlockSpec((QT, 16), lambda qi: (qi, 0)),
        ],
        out_specs=[
            pl.BlockSpec((QT, K_NN), lambda qi: (qi, 0)),
            pl.BlockSpec((QT, K_NN), lambda qi: (qi, 0)),
        ],
        out_shape=[
            jax.ShapeDtypeStruct((nq, K_NN), jnp.float32),
            jax.ShapeDtypeStruct((nq, K_NN), jnp.int32),
        ],
    )(gathered.reshape(jrows, nq, GS), gq)


# speedup vs baseline: 1.5846x; 1.0261x over previous
import functools

import jax
import jax.numpy as jnp
from jax import lax
from jax.experimental import pallas as pl
from jax.experimental.pallas import tpu as pltpu
from jax.experimental.pallas import tpu_sc as plsc

K_NN = 10
GS = 128
QT = 2048
KT = 2048
GT = KT // GS
NEG = float("-inf")
BIG = 2**30


def _p1_body(n_keys, ki_grid, q_ref, k_ref, s_ref, m_ref):
    ki = pl.program_id(0)
    scores = lax.dot_general(
        q_ref[...], k_ref[...], (((1,), (1,)), ((), ())),
        preferred_element_type=jnp.float32)
    for j in range(GT):
        s_ref[j] = scores[:, j * GS:(j + 1) * GS]

    qt = q_ref.shape[0]
    col_iota = lax.broadcasted_iota(jnp.int32, (qt, KT), 1)
    masked = jnp.where(col_iota + ki * KT < n_keys, scores, NEG)
    gmax = jnp.concatenate(
        [jnp.max(masked[:, j * GS:(j + 1) * GS], axis=1, keepdims=True)
         for j in range(GT)], axis=1)
    m_ref[...] = jnp.transpose(gmax)
    del ki_grid


def _p1b_body(nq, m_ref, r_ref, g_ref):
    qi = pl.program_id(0)
    m = m_ref[...]
    mh = m.shape[0]
    qt = m.shape[1]
    gids = lax.broadcasted_iota(jnp.int32, (mh, qt), 0)
    picks = []
    for _ in range(K_NN):
        mx = jnp.max(m, axis=0, keepdims=True)
        g = jnp.min(jnp.where(m == mx, gids, BIG), axis=0, keepdims=True)
        m = jnp.where(gids == g, NEG, m)
        picks.append(g)
    picks.extend([picks[-1]] * (16 - K_NN))
    gmat = jnp.concatenate(picks, axis=0)
    qlane = qi * qt + lax.broadcasted_iota(jnp.int32, (1, qt), 1)
    r_ref[...] = gmat * nq + qlane
    g_ref[...] = jnp.transpose(gmat)


def _sc_gather_body(chunks, ccpr, table_hbm, idx_hbm, out_hbm,
                    idx_v, rows_v, sem0, sem1):
    wid = lax.axis_index("s") * 2 + lax.axis_index("c")
    j0 = (wid * chunks) // ccpr
    cc0 = (wid * chunks) % ccpr
    pltpu.sync_copy(idx_hbm.at[j0, pl.ds(cc0, chunks)], idx_v)
    sems = [sem0, sem1]
    copies = [None, None]
    copies[0] = pltpu.async_copy(
        table_hbm.at[idx_v.at[0]], rows_v.at[0], sems[0])
    for c in range(chunks):
        b = c % 2
        if c + 1 < chunks:
            copies[(c + 1) % 2] = pltpu.async_copy(
                table_hbm.at[idx_v.at[c + 1]], rows_v.at[(c + 1) % 2],
                sems[(c + 1) % 2])
        copies[b].wait()
        pltpu.sync_copy(rows_v.at[b],
                        out_hbm.at[pl.ds((wid * chunks + c) * 128, 128)])


def _p3_body(n_keys, c_ref, g_ref, s_out, i_out):
    cand = jnp.concatenate([c_ref[j] for j in range(K_NN)], axis=1)
    g = g_ref[...]
    qt = g.shape[0]
    lane = lax.broadcasted_iota(jnp.int32, (qt, GS), 1)
    idx = jnp.concatenate(
        [g[:, j:j + 1] * GS + lane for j in range(K_NN)], axis=1)
    cand = jnp.where(idx < n_keys, cand, NEG)
    svals, ivals = [], []
    for _ in range(K_NN):
        mx = jnp.max(cand, axis=1, keepdims=True)
        best = jnp.min(jnp.where(cand == mx, idx, BIG), axis=1,
                       keepdims=True)
        cand = jnp.where(idx == best, NEG, cand)
        svals.append(mx)
        ivals.append(best)
    s_out[...] = jnp.concatenate(svals, axis=1)
    i_out[...] = jnp.concatenate(ivals, axis=1).astype(jnp.int32)


def kernel(queries, keys):
    nq, d = queries.shape
    n_keys = keys.shape[0]
    ki_grid = -(-n_keys // KT)
    kp = ki_grid * KT
    n_groups = kp // GS
    qi_grid = nq // QT

    s_full, m = pl.pallas_call(
        functools.partial(_p1_body, n_keys, ki_grid),
        grid=(ki_grid, qi_grid),
        in_specs=[
            pl.BlockSpec((QT, d), lambda ki, qi: (qi, 0)),
            pl.BlockSpec((KT, d), lambda ki, qi: (ki, 0)),
        ],
        out_specs=[
            pl.BlockSpec((GT, QT, GS), lambda ki, qi: (ki, qi, 0)),
            pl.BlockSpec((GT, QT), lambda ki, qi: (ki, qi)),
        ],
        out_shape=[
            jax.ShapeDtypeStruct((n_groups, nq, GS), jnp.float32),
            jax.ShapeDtypeStruct((n_groups, nq), jnp.float32),
        ],
    )(queries, keys)

    rowids_t, gq = pl.pallas_call(
        functools.partial(_p1b_body, nq),
        grid=(qi_grid,),
        in_specs=[pl.BlockSpec((n_groups, QT), lambda qi: (0, qi))],
        out_specs=[
            pl.BlockSpec((16, QT), lambda qi: (0, qi)),
            pl.BlockSpec((QT, 16), lambda qi: (qi, 0)),
        ],
        out_shape=[
            jax.ShapeDtypeStruct((16, nq), jnp.int32),
            jax.ShapeDtypeStruct((nq, 16), jnp.int32),
        ],
    )(m)

    jrows = 16
    n_rows = nq * jrows
    chunks = n_rows // (32 * 128)
    ccpr = nq // 128
    mesh = plsc.VectorSubcoreMesh(core_axis_name="c", subcore_axis_name="s")
    gathered = pl.kernel(
        functools.partial(_sc_gather_body, chunks, ccpr),
        mesh=mesh,
        out_type=jax.ShapeDtypeStruct((n_rows, GS), jnp.float32),
        scratch_types=[
            pltpu.VMEM((chunks, 128), jnp.int32),
            pltpu.VMEM((2, 128, GS), jnp.float32),
            pltpu.SemaphoreType.DMA,
            pltpu.SemaphoreType.DMA,
        ],
    )(s_full.reshape(n_groups * nq, GS),
      rowids_t.reshape(16, ccpr, 128)[:jrows])

    return pl.pallas_call(
        functools.partial(_p3_body, n_keys),
        grid=(qi_grid,),
        in_specs=[
            pl.BlockSpec((K_NN, QT, GS), lambda qi: (0, qi, 0)),
            pl.BlockSpec((QT, 16), lambda qi: (qi, 0)),
        ],
        out_specs=[
            pl.BlockSpec((QT, K_NN), lambda qi: (qi, 0)),
            pl.BlockSpec((QT, K_NN), lambda qi: (qi, 0)),
        ],
        out_shape=[
            jax.ShapeDtypeStruct((nq, K_NN), jnp.float32),
            jax.ShapeDtypeStruct((nq, K_NN), jnp.int32),
        ],
    )(gathered.reshape(jrows, nq, GS), gq)


# speedup vs baseline: 1.6337x; 1.0310x over previous
import functools

import jax
import jax.numpy as jnp
from jax import lax
from jax.experimental import pallas as pl
from jax.experimental.pallas import tpu as pltpu
from jax.experimental.pallas import tpu_sc as plsc

K_NN = 10
GS = 128
QT = 1024
KT = 4096
GT = KT // GS
NEG = float("-inf")
BIG = 2**30


def _p1_body(n_keys, ki_grid, q_ref, k_ref, s_ref, m_ref):
    ki = pl.program_id(0)
    scores = lax.dot_general(
        q_ref[...], k_ref[...], (((1,), (1,)), ((), ())),
        preferred_element_type=jnp.float32)
    for j in range(GT):
        s_ref[j] = scores[:, j * GS:(j + 1) * GS]

    qt = q_ref.shape[0]
    col_iota = lax.broadcasted_iota(jnp.int32, (qt, KT), 1)
    masked = jnp.where(col_iota + ki * KT < n_keys, scores, NEG)
    gmax = jnp.concatenate(
        [jnp.max(masked[:, j * GS:(j + 1) * GS], axis=1, keepdims=True)
         for j in range(GT)], axis=1)
    m_ref[...] = jnp.transpose(gmax)
    del ki_grid


def _p1b_body(nq, m_ref, r_ref, g_ref):
    qi = pl.program_id(0)
    m = m_ref[...]
    mh = m.shape[0]
    qt = m.shape[1]
    gids = lax.broadcasted_iota(jnp.int32, (mh, qt), 0)
    picks = []
    for _ in range(K_NN):
        mx = jnp.max(m, axis=0, keepdims=True)
        g = jnp.min(jnp.where(m == mx, gids, BIG), axis=0, keepdims=True)
        m = jnp.where(gids == g, NEG, m)
        picks.append(g)
    picks.extend([picks[-1]] * (16 - K_NN))
    gmat = jnp.concatenate(picks, axis=0)
    qlane = qi * qt + lax.broadcasted_iota(jnp.int32, (1, qt), 1)
    r_ref[...] = gmat * nq + qlane
    g_ref[...] = jnp.transpose(gmat)


def _sc_gather_body(chunks, ccpr, table_hbm, idx_hbm, out_hbm,
                    idx_v, rows_v, sem0, sem1):
    wid = lax.axis_index("s") * 2 + lax.axis_index("c")
    j0 = (wid * chunks) // ccpr
    cc0 = (wid * chunks) % ccpr
    pltpu.sync_copy(idx_hbm.at[j0, pl.ds(cc0, chunks)], idx_v)
    sems = [sem0, sem1]
    copies = [None, None]
    copies[0] = pltpu.async_copy(
        table_hbm.at[idx_v.at[0]], rows_v.at[0], sems[0])
    for c in range(chunks):
        b = c % 2
        if c + 1 < chunks:
            copies[(c + 1) % 2] = pltpu.async_copy(
                table_hbm.at[idx_v.at[c + 1]], rows_v.at[(c + 1) % 2],
                sems[(c + 1) % 2])
        copies[b].wait()
        pltpu.sync_copy(rows_v.at[b],
                        out_hbm.at[pl.ds((wid * chunks + c) * 128, 128)])


def _p3_body(n_keys, c_ref, g_ref, s_out, i_out):
    cand = jnp.concatenate([c_ref[j] for j in range(K_NN)], axis=1)
    g = g_ref[...]
    qt = g.shape[0]
    lane = lax.broadcasted_iota(jnp.int32, (qt, GS), 1)
    idx = jnp.concatenate(
        [g[:, j:j + 1] * GS + lane for j in range(K_NN)], axis=1)
    cand = jnp.where(idx < n_keys, cand, NEG)
    svals, ivals = [], []
    for _ in range(K_NN):
        mx = jnp.max(cand, axis=1, keepdims=True)
        best = jnp.min(jnp.where(cand == mx, idx, BIG), axis=1,
                       keepdims=True)
        cand = jnp.where(idx == best, NEG, cand)
        svals.append(mx)
        ivals.append(best)
    s_out[...] = jnp.concatenate(svals, axis=1)
    i_out[...] = jnp.concatenate(ivals, axis=1).astype(jnp.int32)


def kernel(queries, keys):
    nq, d = queries.shape
    n_keys = keys.shape[0]
    ki_grid = -(-n_keys // KT)
    kp = ki_grid * KT
    n_groups = kp // GS
    qi_grid = nq // QT

    s_full, m = pl.pallas_call(
        functools.partial(_p1_body, n_keys, ki_grid),
        grid=(ki_grid, qi_grid),
        in_specs=[
            pl.BlockSpec((QT, d), lambda ki, qi: (qi, 0)),
            pl.BlockSpec((KT, d), lambda ki, qi: (ki, 0)),
        ],
        out_specs=[
            pl.BlockSpec((GT, QT, GS), lambda ki, qi: (ki, qi, 0)),
            pl.BlockSpec((GT, QT), lambda ki, qi: (ki, qi)),
        ],
        out_shape=[
            jax.ShapeDtypeStruct((n_groups, nq, GS), jnp.float32),
            jax.ShapeDtypeStruct((n_groups, nq), jnp.float32),
        ],
    )(queries, keys)

    rowids_t, gq = pl.pallas_call(
        functools.partial(_p1b_body, nq),
        grid=(qi_grid,),
        in_specs=[pl.BlockSpec((n_groups, QT), lambda qi: (0, qi))],
        out_specs=[
            pl.BlockSpec((16, QT), lambda qi: (0, qi)),
            pl.BlockSpec((QT, 16), lambda qi: (qi, 0)),
        ],
        out_shape=[
            jax.ShapeDtypeStruct((16, nq), jnp.int32),
            jax.ShapeDtypeStruct((nq, 16), jnp.int32),
        ],
    )(m)

    jrows = 16
    n_rows = nq * jrows
    chunks = n_rows // (32 * 128)
    ccpr = nq // 128
    mesh = plsc.VectorSubcoreMesh(core_axis_name="c", subcore_axis_name="s")
    gathered = pl.kernel(
        functools.partial(_sc_gather_body, chunks, ccpr),
        mesh=mesh,
        out_type=jax.ShapeDtypeStruct((n_rows, GS), jnp.float32),
        scratch_types=[
            pltpu.VMEM((chunks, 128), jnp.int32),
            pltpu.VMEM((2, 128, GS), jnp.float32),
            pltpu.SemaphoreType.DMA,
            pltpu.SemaphoreType.DMA,
        ],
    )(s_full.reshape(n_groups * nq, GS),
      rowids_t.reshape(16, ccpr, 128)[:jrows])

    return pl.pallas_call(
        functools.partial(_p3_body, n_keys),
        grid=(qi_grid,),
        in_specs=[
            pl.BlockSpec((K_NN, QT, GS), lambda qi: (0, qi, 0)),
            pl.BlockSpec((QT, 16), lambda qi: (qi, 0)),
        ],
        out_specs=[
            pl.BlockSpec((QT, K_NN), lambda qi: (qi, 0)),
            pl.BlockSpec((QT, K_NN), lambda qi: (qi, 0)),
        ],
        out_shape=[
            jax.ShapeDtypeStruct((nq, K_NN), jnp.float32),
            jax.ShapeDtypeStruct((nq, K_NN), jnp.int32),
        ],
    )(gathered.reshape(jrows, nq, GS), gq)
